# baseline (speedup 1.0000x reference)
import functools

import jax
import jax.numpy as jnp
from jax import lax
from jax.experimental import pallas as pl
from jax.experimental.pallas import tpu as pltpu
from jax.experimental.pallas import tpu_sc as plsc

BATCH = 16384
D = 64
W = 2 * D
L = 16
NW = 32
BPW = BATCH // NW
CH = 128
NCH = BPW // CH
TB = 4000

_mesh = plsc.VectorSubcoreMesh(core_axis_name="c", subcore_axis_name="s")


def _dup_body(x_ref, o_ref):
    x = x_ref[...]
    o_ref[...] = jnp.concatenate([x, x], axis=1)


def _widen(table):
    n = table.shape[0]
    return pl.pallas_call(
        _dup_body,
        grid=(n // TB,),
        in_specs=[pl.BlockSpec((TB, D), lambda i: (i, 0))],
        out_specs=pl.BlockSpec((TB, W), lambda i: (i, 0)),
        out_shape=jax.ShapeDtypeStruct((n, W), jnp.float32),
    )(table)


@functools.partial(
    pl.kernel,
    mesh=_mesh,
    out_type=(
        jax.ShapeDtypeStruct((BATCH,), jnp.float32),
        jax.ShapeDtypeStruct((BATCH,), jnp.float32),
    ),
    scratch_types=[
        pltpu.VMEM((BPW,), jnp.int32),
        pltpu.VMEM((BPW,), jnp.int32),
        pltpu.VMEM((BPW,), jnp.int32),
        pltpu.VMEM((CH, W), jnp.float32),
        pltpu.VMEM((CH, W), jnp.float32),
        pltpu.VMEM((CH, W), jnp.float32),
        pltpu.VMEM((BPW,), jnp.float32),
        pltpu.VMEM((BPW,), jnp.float32),
        pltpu.SemaphoreType.DMA,
    ],
    compiler_params=pltpu.CompilerParams(needs_layout_passes=False),
)
def _bprmf_sc(user_hbm, itemi_hbm, itemj_hbm, ut_hbm, it_hbm,
              out_i, out_j,
              uix, iix, jix, bu, bi, bj, oi, oj, sem):
    wid = lax.axis_index("s") * 2 + lax.axis_index("c")
    base = wid * BPW

    pltpu.sync_copy(user_hbm.at[pl.ds(base, BPW)], uix)
    pltpu.sync_copy(itemi_hbm.at[pl.ds(base, BPW)], iix)
    pltpu.sync_copy(itemj_hbm.at[pl.ds(base, BPW)], jix)

    iota = jnp.arange(L, dtype=jnp.int32)

    def chunk(c, carry):
        cix = pl.ds(c * CH, CH)
        cpu = pltpu.async_copy(ut_hbm.at[uix.at[cix]], bu, sem)
        cpi = pltpu.async_copy(it_hbm.at[iix.at[cix]], bi, sem)
        cpj = pltpu.async_copy(it_hbm.at[jix.at[cix]], bj, sem)
        cpu.wait()
        cpi.wait()
        cpj.wait()

        def body(g, carry):
            ivec = g * L + iota
            acc_i = jnp.zeros((L,), jnp.float32)
            acc_j = jnp.zeros((L,), jnp.float32)
            for d in range(D):
                dvec = jnp.full((L,), d, dtype=jnp.int32)
                u = plsc.load_gather(bu, [ivec, dvec])
                acc_i = acc_i + u * plsc.load_gather(bi, [ivec, dvec])
                acc_j = acc_j + u * plsc.load_gather(bj, [ivec, dvec])
            off = pl.multiple_of(c * CH + g * L, L)
            oi[pl.ds(off, L)] = acc_i
            oj[pl.ds(off, L)] = acc_j
            return carry

        lax.fori_loop(0, CH // L, body, 0)
        return carry

    lax.fori_loop(0, NCH, chunk, 0)

    pltpu.sync_copy(oi, out_i.at[pl.ds(base, BPW)])
    pltpu.sync_copy(oj, out_j.at[pl.ds(base, BPW)])


def kernel(user, item_i, item_j, user_table, item_table):
    ut_wide = _widen(user_table)
    it_wide = _widen(item_table)
    return _bprmf_sc(user.astype(jnp.int32), item_i.astype(jnp.int32),
                     item_j.astype(jnp.int32), ut_wide, it_wide)


# speedup vs baseline: 1.0670x; 1.0670x over previous
import functools

import jax
import jax.numpy as jnp
from jax import lax
from jax.experimental import pallas as pl
from jax.experimental.pallas import tpu as pltpu
from jax.experimental.pallas import tpu_sc as plsc

BATCH = 16384
D = 64
W = 2 * D
L = 16
NW = 32
BPW = BATCH // NW
CH = 128
NCH = BPW // CH
TB = 4000

_mesh = plsc.VectorSubcoreMesh(core_axis_name="c", subcore_axis_name="s")


def _dup_body(x_ref, o_ref):
    x = x_ref[...]
    o_ref[...] = jnp.concatenate([x, x], axis=1)


def _widen(table):
    n = table.shape[0]
    return pl.pallas_call(
        _dup_body,
        grid=(n // TB,),
        in_specs=[pl.BlockSpec((TB, D), lambda i: (i, 0))],
        out_specs=pl.BlockSpec((TB, W), lambda i: (i, 0)),
        out_shape=jax.ShapeDtypeStruct((n, W), jnp.float32),
    )(table)


@functools.partial(
    pl.kernel,
    mesh=_mesh,
    out_type=(
        jax.ShapeDtypeStruct((BATCH,), jnp.float32),
        jax.ShapeDtypeStruct((BATCH,), jnp.float32),
    ),
    scratch_types=[
        pltpu.VMEM((BPW,), jnp.int32),
        pltpu.VMEM((BPW,), jnp.int32),
        pltpu.VMEM((BPW,), jnp.int32),
        pltpu.VMEM((CH, W), jnp.float32),
        pltpu.VMEM((CH, W), jnp.float32),
        pltpu.VMEM((CH, W), jnp.float32),
        pltpu.VMEM((BPW,), jnp.float32),
        pltpu.VMEM((BPW,), jnp.float32),
        pltpu.SemaphoreType.DMA,
    ],
    compiler_params=pltpu.CompilerParams(needs_layout_passes=False),
)
def _bprmf_sc(user_hbm, itemi_hbm, itemj_hbm, ut_hbm, it_hbm,
              out_i, out_j,
              uix, iix, jix, bu, bi, bj, oi, oj, sem):
    wid = lax.axis_index("s") * 2 + lax.axis_index("c")
    base = wid * BPW

    pltpu.sync_copy(user_hbm.at[pl.ds(base, BPW)], uix)
    pltpu.sync_copy(itemi_hbm.at[pl.ds(base, BPW)], iix)
    pltpu.sync_copy(itemj_hbm.at[pl.ds(base, BPW)], jix)

    iota = jnp.arange(L, dtype=jnp.int32)

    def chunk(c, carry):
        cix = pl.ds(c * CH, CH)
        cpu = pltpu.async_copy(ut_hbm.at[uix.at[cix]], bu, sem)
        cpi = pltpu.async_copy(it_hbm.at[iix.at[cix]], bi, sem)
        cpj = pltpu.async_copy(it_hbm.at[jix.at[cix]], bj, sem)
        cpu.wait()
        cpi.wait()
        cpj.wait()

        def body(g, carry):
            ivec = g * L + iota
            acc_i = jnp.zeros((L,), jnp.float32)
            acc_j = jnp.zeros((L,), jnp.float32)
            for d in range(D):
                dvec = jnp.full((L,), d, dtype=jnp.int32)
                u = plsc.load_gather(bu, [ivec, dvec])
                acc_i = acc_i + u * plsc.load_gather(bi, [ivec, dvec])
                acc_j = acc_j + u * plsc.load_gather(bj, [ivec, dvec])
            off = pl.multiple_of(c * CH + g * L, L)
            oi[pl.ds(off, L)] = acc_i
            oj[pl.ds(off, L)] = acc_j
            return carry

        lax.fori_loop(0, CH // L, body, 0)
        return carry

    lax.fori_loop(0, NCH, chunk, 0)

    pltpu.sync_copy(oi, out_i.at[pl.ds(base, BPW)])
    pltpu.sync_copy(oj, out_j.at[pl.ds(base, BPW)])


def kernel(user, item_i, item_j, user_table, item_table):
    ut_wide = jnp.concatenate([user_table, user_table], axis=1)
    it_wide = jnp.concatenate([item_table, item_table], axis=1)
    return _bprmf_sc(user.astype(jnp.int32), item_i.astype(jnp.int32),
                     item_j.astype(jnp.int32), ut_wide, it_wide)


# speedup vs baseline: 1.6371x; 1.5343x over previous
import functools

import jax
import jax.numpy as jnp
from jax import lax
from jax.experimental import pallas as pl
from jax.experimental.pallas import tpu as pltpu
from jax.experimental.pallas import tpu_sc as plsc

BATCH = 16384
D = 64
W = 128
L = 16
NW = 32
BPW = BATCH // NW
CH = 128
NCH = BPW // CH

_mesh = plsc.VectorSubcoreMesh(core_axis_name="c", subcore_axis_name="s")


@functools.partial(
    pl.kernel,
    mesh=_mesh,
    out_type=jax.ShapeDtypeStruct((BATCH, W), jnp.float32),
    scratch_types=[
        pltpu.VMEM((BPW,), jnp.int32),
        pltpu.VMEM((BPW, W), jnp.float32),
        pltpu.SemaphoreType.DMA,
    ],
    compiler_params=pltpu.CompilerParams(needs_layout_passes=False),
)
def _gather_user(user_hbm, ut_hbm, out_rows, uix, bu, sem):
    wid = lax.axis_index("s") * 2 + lax.axis_index("c")
    base = wid * BPW

    pltpu.sync_copy(user_hbm.at[pl.ds(base, BPW)], uix)

    def issue(g, carry):
        off = pl.multiple_of(g * L, L)
        vec = uix[pl.ds(off, L)]
        for l in range(L):
            pltpu.async_copy(
                ut_hbm.at[vec[l]], bu.at[g * L + l, pl.ds(0, D)], sem
            )
        return carry

    lax.fori_loop(0, BPW // L, issue, 0)

    def drain(g, carry):
        for _ in range(L):
            pltpu.make_async_copy(
                ut_hbm.at[0], bu.at[0, pl.ds(0, D)], sem
            ).wait()
        return carry

    lax.fori_loop(0, BPW // L, drain, 0)

    pltpu.sync_copy(bu, out_rows.at[pl.ds(base, BPW)])


@functools.partial(
    pl.kernel,
    mesh=_mesh,
    out_type=(
        jax.ShapeDtypeStruct((BATCH,), jnp.float32),
        jax.ShapeDtypeStruct((BATCH,), jnp.float32),
    ),
    scratch_types=[
        pltpu.VMEM((NCH, CH), jnp.int32),
        pltpu.VMEM((NCH, CH), jnp.int32),
        pltpu.VMEM((BPW, D), jnp.float32),
        pltpu.VMEM((BPW, D), jnp.float32),
        pltpu.VMEM((BPW, D), jnp.float32),
        pltpu.VMEM((BPW,), jnp.float32),
        pltpu.VMEM((BPW,), jnp.float32),
        pltpu.SemaphoreType.DMA,
    ],
    compiler_params=pltpu.CompilerParams(
        use_tc_tiling_on_sc=False, needs_layout_passes=False
    ),
)
def _dots(itemi_hbm, itemj_hbm, urows_hbm, it_hbm,
          out_i, out_j,
          iix, jix, bu, bi, bj, oi, oj, sem):
    wid = lax.axis_index("s") * 2 + lax.axis_index("c")
    base = wid * BPW

    pltpu.sync_copy(itemi_hbm.at[wid], iix)
    pltpu.sync_copy(itemj_hbm.at[wid], jix)

    copies = [pltpu.async_copy(
        urows_hbm.at[pl.ds(base, BPW), pl.ds(0, D)], bu, sem)]
    for k in range(NCH):
        dst = pl.ds(k * CH, CH)
        copies.append(pltpu.async_copy(it_hbm.at[iix.at[k]], bi.at[dst], sem))
        copies.append(pltpu.async_copy(it_hbm.at[jix.at[k]], bj.at[dst], sem))
    for cp in copies:
        cp.wait()

    iota = jnp.arange(L, dtype=jnp.int32)

    def body(g, carry):
        rowids = g * L + iota
        acc_i = jnp.zeros((L,), jnp.float32)
        acc_j = jnp.zeros((L,), jnp.float32)
        for d in range(D):
            colids = jnp.full((L,), d, dtype=jnp.int32)
            u = plsc.load_gather(bu, [rowids, colids])
            acc_i = acc_i + u * plsc.load_gather(bi, [rowids, colids])
            acc_j = acc_j + u * plsc.load_gather(bj, [rowids, colids])
        off = pl.multiple_of(g * L, L)
        oi[pl.ds(off, L)] = acc_i
        oj[pl.ds(off, L)] = acc_j
        return carry

    lax.fori_loop(0, BPW // L, body, 0)

    pltpu.sync_copy(oi, out_i.at[pl.ds(base, BPW)])
    pltpu.sync_copy(oj, out_j.at[pl.ds(base, BPW)])


def kernel(user, item_i, item_j, user_table, item_table):
    urows = _gather_user(user.astype(jnp.int32), user_table)
    itemi_r = item_i.astype(jnp.int32).reshape(NW, NCH, CH)
    itemj_r = item_j.astype(jnp.int32).reshape(NW, NCH, CH)
    return _dots(itemi_r, itemj_r, urows, item_table)


# speedup vs baseline: 2.0205x; 1.2342x over previous
import functools

import jax
import jax.numpy as jnp
from jax import lax
from jax.experimental import pallas as pl
from jax.experimental.pallas import tpu as pltpu
from jax.experimental.pallas import tpu_sc as plsc

BATCH = 16384
D = 64
L = 16
NW = 32
BPW = BATCH // NW
HB = BPW // 2
NG = BPW // L
HG = NG // 2

_mesh = plsc.VectorSubcoreMesh(core_axis_name="c", subcore_axis_name="s")


@functools.partial(
    pl.kernel,
    mesh=_mesh,
    out_type=(
        jax.ShapeDtypeStruct((BATCH,), jnp.float32),
        jax.ShapeDtypeStruct((BATCH,), jnp.float32),
    ),
    scratch_types=[
        pltpu.VMEM((BPW,), jnp.int32),
        pltpu.VMEM((BPW,), jnp.int32),
        pltpu.VMEM((BPW,), jnp.int32),
        pltpu.VMEM((HB, 2 * D), jnp.float32),
        pltpu.VMEM((HB, 2 * D), jnp.float32),
        pltpu.VMEM((HB, 2 * D), jnp.float32),
        pltpu.VMEM((BPW,), jnp.float32),
        pltpu.VMEM((BPW,), jnp.float32),
        pltpu.SemaphoreType.DMA,
    ],
    compiler_params=pltpu.CompilerParams(needs_layout_passes=False),
)
def _bprmf_sc(user_hbm, itemi_hbm, itemj_hbm, ut_hbm, it_hbm,
              out_i, out_j,
              uix, iix, jix, bu, bi, bj, oi, oj, sem):
    wid = lax.axis_index("s") * 2 + lax.axis_index("c")
    base = wid * BPW

    pltpu.sync_copy(user_hbm.at[pl.ds(base, BPW)], uix)
    pltpu.sync_copy(itemi_hbm.at[pl.ds(base, BPW)], iix)
    pltpu.sync_copy(itemj_hbm.at[pl.ds(base, BPW)], jix)

    for side in range(2):
        half = pl.ds(side * D, D)

        def issue(g, carry):
            off = pl.multiple_of(side * HB + g * L, L)
            uvec = uix[pl.ds(off, L)]
            ivec = iix[pl.ds(off, L)]
            jvec = jix[pl.ds(off, L)]
            for l in range(L):
                row = g * L + l
                pltpu.async_copy(ut_hbm.at[uvec[l]], bu.at[row, half], sem)
                pltpu.async_copy(it_hbm.at[ivec[l]], bi.at[row, half], sem)
                pltpu.async_copy(it_hbm.at[jvec[l]], bj.at[row, half], sem)
            return carry

        lax.fori_loop(0, HG, issue, 0)

    def drain(g, carry):
        for _ in range(3 * L):
            pltpu.make_async_copy(
                ut_hbm.at[0], bu.at[0, pl.ds(0, D)], sem
            ).wait()
        return carry

    lax.fori_loop(0, NG, drain, 0)

    iota = jnp.arange(L, dtype=jnp.int32)

    for side in range(2):

        def body(g, carry):
            rowids = g * L + iota
            acc_i = jnp.zeros((L,), jnp.float32)
            acc_j = jnp.zeros((L,), jnp.float32)
            for d in range(D):
                colids = jnp.full((L,), side * D + d, dtype=jnp.int32)
                u = plsc.load_gather(bu, [rowids, colids])
                acc_i = acc_i + u * plsc.load_gather(bi, [rowids, colids])
                acc_j = acc_j + u * plsc.load_gather(bj, [rowids, colids])
            off = pl.multiple_of(side * HB + g * L, L)
            oi[pl.ds(off, L)] = acc_i
            oj[pl.ds(off, L)] = acc_j
            return carry

        lax.fori_loop(0, HG, body, 0)

    pltpu.sync_copy(oi, out_i.at[pl.ds(base, BPW)])
    pltpu.sync_copy(oj, out_j.at[pl.ds(base, BPW)])


def kernel(user, item_i, item_j, user_table, item_table):
    return _bprmf_sc(user.astype(jnp.int32), item_i.astype(jnp.int32),
                     item_j.astype(jnp.int32), user_table, item_table)
